# baseline (speedup 1.0000x reference)
import functools

import jax
import jax.numpy as jnp
from jax.experimental import pallas as pl

_DEPTH = 10
_DIM = 1024
_NB_SPLIT = 2**_DEPTH - 1
_NB_NODES = 2**(_DEPTH + 1) - 1


def _expand(lvl, score, L):
    rows = jax.lax.broadcasted_iota(jnp.int32, (L, 2 * L), 0)
    cols = jax.lax.broadcasted_iota(jnp.int32, (L, 2 * L), 1)
    hit = cols // 2 == rows
    r = jnp.where(hit, 1.0, 0.0)
    rs = jnp.where(hit, jnp.where(cols % 2 == 0, 1.0, -1.0), 0.0)
    rep_parent = jnp.dot(lvl, r, preferred_element_type=jnp.float32)
    rep_score = jnp.dot(score, rs, preferred_element_type=jnp.float32)
    return jnp.minimum(rep_parent, rep_score)


def _tree_body(x_ref, at_ref, o_ref, *, tb):
    x = x_ref[:]
    xa_a = jnp.dot(x, at_ref[:, 0:256], preferred_element_type=jnp.float32)
    xa_b = jnp.dot(x, at_ref[:, 256:512], preferred_element_type=jnp.float32)
    xa_c = jnp.dot(x, at_ref[:, 512:768], preferred_element_type=jnp.float32)
    xa_d = jnp.dot(x, at_ref[:, 768:1024], preferred_element_type=jnp.float32)

    o_ref[:, 0:1] = jnp.ones((tb, 1), jnp.float32)
    lvl = jnp.ones((tb, 1), jnp.float32)
    for d in range(_DEPTH):
        L = 1 << d
        if d < 8:
            score = xa_a[:, L - 1:2 * L - 1]
        elif d == 8:
            score = xa_b
        else:
            score = jnp.concatenate([xa_c, xa_d], axis=1)
        lvl = _expand(lvl, score, L)
        o_ref[:, 2 * L - 1:4 * L - 1] = jnp.maximum(lvl, 0.0)


@jax.jit
def kernel(x, A):
    b, dim = x.shape
    a_t = A.T
    a_t = jnp.concatenate(
        [
            a_t[:, 0:255],
            jnp.zeros((dim, 1), A.dtype),
            a_t[:, 255:1023],
        ],
        axis=1,
    )
    tb = 1024
    return pl.pallas_call(
        functools.partial(_tree_body, tb=tb),
        grid=(b // tb,),
        in_specs=[
            pl.BlockSpec((tb, dim), lambda i: (i, 0)),
            pl.BlockSpec((dim, _DIM), lambda i: (0, 0)),
        ],
        out_specs=pl.BlockSpec((tb, _NB_NODES), lambda i: (i, 0)),
        out_shape=jax.ShapeDtypeStruct((b, _NB_NODES), jnp.float32),
    )(x, a_t)


# speedup vs baseline: 1.0206x; 1.0206x over previous
import functools

import jax
import jax.numpy as jnp
from jax.experimental import pallas as pl

_DEPTH = 10
_DIM = 1024
_NB_SPLIT = 2**_DEPTH - 1
_NB_NODES = 2**(_DEPTH + 1) - 1


def _expand(lvl, score, L):
    rows = jax.lax.broadcasted_iota(jnp.int32, (L, 2 * L), 0)
    cols = jax.lax.broadcasted_iota(jnp.int32, (L, 2 * L), 1)
    cl = (cols + 1) % (2 * L)
    parent = cl // 2
    r = jnp.where(rows == (parent - 1) % L, 1.0, 0.0)
    rs = jnp.where(rows == parent,
                   jnp.where(cl % 2 == 0, 1.0, -1.0), 0.0)
    rep_parent = jnp.dot(lvl, r, preferred_element_type=jnp.float32)
    rep_score = jnp.dot(score, rs, preferred_element_type=jnp.float32)
    return jnp.maximum(jnp.minimum(rep_parent, rep_score), 0.0)


def _tree_body(x_ref, at_ref, o_ref, *, tb):
    x = x_ref[:]
    xa_a = jnp.dot(x, at_ref[:, 0:256], preferred_element_type=jnp.float32)
    xa_b = jnp.dot(x, at_ref[:, 256:512], preferred_element_type=jnp.float32)
    xa_c = jnp.dot(x, at_ref[:, 512:768], preferred_element_type=jnp.float32)
    xa_d = jnp.dot(x, at_ref[:, 768:1024], preferred_element_type=jnp.float32)

    o_ref[:, 0:1] = jnp.ones((tb, 1), jnp.float32)
    lvl = jnp.ones((tb, 1), jnp.float32)
    for d in range(_DEPTH):
        L = 1 << d
        if d < 8:
            score = xa_a[:, L - 1:2 * L - 1]
        elif d == 8:
            score = xa_b
        else:
            score = jnp.concatenate([xa_c, xa_d], axis=1)
        lvl = _expand(lvl, score, L)
        o_ref[:, 2 * L:4 * L - 1] = lvl[:, 0:2 * L - 1]
        o_ref[:, 2 * L - 1:2 * L] = lvl[:, 2 * L - 1:2 * L]


@jax.jit
def kernel(x, A):
    b, dim = x.shape
    a_t = A.T
    a_t = jnp.concatenate(
        [
            a_t[:, 0:255],
            jnp.zeros((dim, 1), A.dtype),
            a_t[:, 255:1023],
        ],
        axis=1,
    )
    tb = 1024
    return pl.pallas_call(
        functools.partial(_tree_body, tb=tb),
        grid=(b // tb,),
        in_specs=[
            pl.BlockSpec((tb, dim), lambda i: (i, 0)),
            pl.BlockSpec((dim, _DIM), lambda i: (0, 0)),
        ],
        out_specs=pl.BlockSpec((tb, _NB_NODES), lambda i: (i, 0)),
        out_shape=jax.ShapeDtypeStruct((b, _NB_NODES), jnp.float32),
    )(x, a_t)
